# baseline (speedup 1.0000x reference)
import math

import jax
import jax.numpy as jnp
from jax.experimental import pallas as pl

B = 16
C = 256
HW = 256
N = 512
NQ = 520
GB = 4
D = 257
FF = 1028
NC = 150
NS = 17
NEG = float(jnp.finfo(jnp.float32).min)
INV_SQRT_D = 1.0 / math.sqrt(257.0)


def _fused_body(ss_ref, so_ref, pos_ref, *refs):
    wrefs = refs[:-2]
    rows_ref, hv_ref = refs[-2:]
    f32 = jnp.float32

    def pool(xc):
        t = xc.T
        a = jnp.max(t.reshape(16, 2, 32, C), axis=1)
        b = jnp.max(a.reshape(16, 16, 2, C), axis=2)
        return b.reshape(HW, C)

    def ln(z, g, c):
        mu = jnp.mean(z, axis=1, keepdims=True)
        zc = z - mu
        var = jnp.mean(zc * zc, axis=1, keepdims=True)
        return zc * jax.lax.rsqrt(var + 1e-5) * g + c

    pos = pos_ref[...]
    zlane = jnp.zeros((HW, D - C), f32)
    for bi in range(GB):
        ps = pool(ss_ref[bi])
        po = pool(so_ref[bi])
        hs = jnp.sum(ps, axis=1, keepdims=True)
        ho = jnp.sum(po, axis=1, keepdims=True)
        hv_ref[bi, 0:1, 0:HW] = hs.reshape(1, HW)
        hv_ref[bi, 0:1, HW:N] = ho.reshape(1, HW)
        vs = (hs != 0.0).astype(f32)
        vo = (ho != 0.0).astype(f32)
        vm = jnp.concatenate([vs.reshape(1, HW), vo.reshape(1, HW)], axis=1)

        xs = (jnp.concatenate([ps, zlane], axis=1) + pos) * vs
        xo = (jnp.concatenate([po, zlane], axis=1) + pos) * vo
        x = jnp.concatenate([xs, xo, jnp.zeros((NQ - N, D), f32)], axis=0)

        for li in range(2):
            (wq, wk, wv, wo, g1, c1, g2, c2, w1, bb1, w2, bb2) = (
                r[...] for r in wrefs[12 * li:12 * li + 12])
            h = ln(x, g1, c1)
            q = jnp.dot(h, wq, preferred_element_type=f32)
            k = jnp.dot(h[0:N], wk, preferred_element_type=f32)
            v = jnp.dot(h[0:N], wv, preferred_element_type=f32)
            s = jax.lax.dot_general(q, k, (((1,), (1,)), ((), ())),
                                    preferred_element_type=f32) * INV_SQRT_D
            s = jnp.where(vm > 0.0, s, NEG)
            m = jnp.max(s, axis=1, keepdims=True)
            e = jnp.exp(s - m)
            p = e / jnp.sum(e, axis=1, keepdims=True)
            o = jnp.dot(p, v, preferred_element_type=f32)
            x = x + jnp.dot(o, wo, preferred_element_type=f32)
            h2 = ln(x, g2, c2)
            a = jax.nn.gelu(jnp.dot(h2, w1, preferred_element_type=f32) + bb1)
            x = x + jnp.dot(a, w2, preferred_element_type=f32) + bb2

        rows_ref[bi] = x[N - 1:N + 1, :]


def _fused(ss, so, pos_tok, ws):
    full = lambda a: pl.BlockSpec(a.shape, lambda b: (0,) * a.ndim)
    return pl.pallas_call(
        _fused_body,
        grid=(B // GB,),
        in_specs=[
            pl.BlockSpec((GB, C, 1024), lambda b: (b, 0, 0)),
            pl.BlockSpec((GB, C, 1024), lambda b: (b, 0, 0)),
            pl.BlockSpec((HW, D), lambda b: (0, 0)),
        ] + [full(w) for w in ws],
        out_specs=[
            pl.BlockSpec((GB, 2, D), lambda b: (b, 0, 0)),
            pl.BlockSpec((GB, 1, N), lambda b: (b, 0, 0)),
        ],
        out_shape=[
            jax.ShapeDtypeStruct((B, 2, D), jnp.float32),
            jax.ShapeDtypeStruct((B, 1, N), jnp.float32),
        ],
    )(ss, so, pos_tok, *ws)


def _heads_body(rows_ref, hv_ref, cs_ref, co_ref, ssc_ref, soc_ref,
                fw_ref, fb_ref, hw_ref, hb_ref, out_ref):
    f32 = jnp.float32
    hv = hv_ref[:, 0, :]
    all_valid = jnp.min(jnp.where(hv != 0.0, 1.0, 0.0), axis=1,
                        keepdims=True)
    hidden = (rows_ref[:, 0, :] * all_valid
              + rows_ref[:, 1, :] * (1.0 - all_valid))

    cs = cs_ref[...].reshape(B, 1)
    co = co_ref[...].reshape(B, 1)
    lane = jax.lax.broadcasted_iota(jnp.int32, (B, NC), 1)
    oh1 = (lane == cs).astype(f32)
    oh2 = (lane == co).astype(f32)
    hc = jnp.concatenate([hidden, oh1, oh2, ssc_ref[...], soc_ref[...]],
                         axis=1)

    pred = jnp.maximum(
        jnp.dot(hc, fw_ref[...], preferred_element_type=f32) + fb_ref[...],
        0.0)
    z = jnp.dot(pred, hw_ref[...], preferred_element_type=f32) + hb_ref[...]

    def lsm(seg):
        m = jnp.max(seg, axis=1, keepdims=True)
        lse = jnp.log(jnp.sum(jnp.exp(seg - m), axis=1, keepdims=True)) + m
        return seg - lse

    sup = lsm(z[:, 50:53])
    r1 = lsm(z[:, 0:15]) + sup[:, 0:1]
    r2 = lsm(z[:, 15:26]) + sup[:, 1:2]
    r3 = lsm(z[:, 26:50]) + sup[:, 2:3]
    conn = z[:, 53:54]
    out_ref[...] = jnp.concatenate(
        [r1, r2, r3, sup, conn, jnp.zeros((B, 128 - 54), f32)], axis=1)


def _heads(rows, hv, cs, co, ssc, soc, fw, fb, hwts, hb):
    args = (rows, hv, cs, co, ssc, soc, fw, fb, hwts, hb)
    full = lambda a: pl.BlockSpec(a.shape, lambda: (0,) * a.ndim)
    return pl.pallas_call(
        _heads_body,
        in_specs=[full(a) for a in args],
        out_specs=pl.BlockSpec((B, 128), lambda: (0, 0)),
        out_shape=jax.ShapeDtypeStruct((B, 128), jnp.float32),
    )(*args)


def kernel(src_sub, src_obj, cat_sub, cat_obj, scat_sub, scat_obj, params):
    f32 = jnp.float32
    pe = params['pe']
    pos_tok = jnp.transpose(pe[:16, :16], (1, 0, 2)).reshape(HW, 255)
    tail = (jnp.arange(D - 255)[None, :] == 1).astype(f32)
    pos_tok = jnp.concatenate(
        [pos_tok, jnp.broadcast_to(tail, (HW, D - 255))], axis=1)

    L = params['layers']
    keys = ('wq', 'wk', 'wv', 'wo', 'ln1_g', 'ln1_b', 'ln2_g', 'ln2_b',
            'w1', 'b1', 'w2', 'b2')
    ws = []
    for li in range(2):
        for key in keys:
            w = L[li][key]
            ws.append(w.reshape(1, -1) if w.ndim == 1 else w)

    rows, hv = _fused(src_sub.reshape(B, C, 1024),
                      src_obj.reshape(B, C, 1024), pos_tok, ws)

    hwts = jnp.concatenate([params['fc21_w'], params['fc22_w'],
                            params['fc23_w'], params['fc3_w'],
                            params['fc4_w']], axis=1)
    hb = jnp.concatenate([params['fc21_b'], params['fc22_b'],
                          params['fc23_b'], params['fc3_b'],
                          params['fc4_b']]).reshape(1, 54)

    out = _heads(rows, hv, cat_sub.reshape(1, B), cat_obj.reshape(1, B),
                 scat_sub, scat_obj, params['fc1_w'],
                 params['fc1_b'].reshape(1, D), hwts, hb)
    return (out[:, 0:15], out[:, 15:26], out[:, 26:50],
            out[:, 50:53], out[:, 53:54])


# speedup vs baseline: 1.2748x; 1.2748x over previous
import math

import jax
import jax.numpy as jnp
from jax.experimental import pallas as pl

B = 16
C = 256
HW = 256
N = 512
NQ = 520
D = 257
FF = 1028
NC = 150
NS = 17
NEG = float(jnp.finfo(jnp.float32).min)
INV_SQRT_D = 1.0 / math.sqrt(257.0)


def _fused_body(ss_ref, so_ref, pos_ref, *refs):
    wrefs = refs[:-2]
    rows_ref, hv_ref = refs[-2:]
    f32 = jnp.float32

    def pool(xc):
        t = xc.T
        a = jnp.max(t.reshape(16, 2, 32, C), axis=1)
        b = jnp.max(a.reshape(16, 16, 2, C), axis=2)
        return b.reshape(HW, C)

    ps = pool(ss_ref[0])
    po = pool(so_ref[0])
    hs = jnp.sum(ps, axis=1, keepdims=True)
    ho = jnp.sum(po, axis=1, keepdims=True)
    hv_ref[0, 0:1, 0:HW] = hs.reshape(1, HW)
    hv_ref[0, 0:1, HW:N] = ho.reshape(1, HW)
    vs = (hs != 0.0).astype(f32)
    vo = (ho != 0.0).astype(f32)
    vm = jnp.concatenate([vs.reshape(1, HW), vo.reshape(1, HW)], axis=1)

    pos = pos_ref[...]
    zlane = jnp.zeros((HW, D - C), f32)
    xs = (jnp.concatenate([ps, zlane], axis=1) + pos) * vs
    xo = (jnp.concatenate([po, zlane], axis=1) + pos) * vo
    x = jnp.concatenate([xs, xo, jnp.zeros((NQ - N, D), f32)], axis=0)

    def ln(z, g, c):
        mu = jnp.mean(z, axis=1, keepdims=True)
        zc = z - mu
        var = jnp.mean(zc * zc, axis=1, keepdims=True)
        return zc * jax.lax.rsqrt(var + 1e-5) * g + c

    def attn_ffn(x, qrows, wl):
        (wq, wk, wv, wo, g1, c1, g2, c2, w1, bb1, w2, bb2) = wl
        h = ln(x, g1, c1)
        q = jnp.dot(h[qrows], wq, preferred_element_type=f32)
        k = jnp.dot(h[0:N], wk, preferred_element_type=f32)
        v = jnp.dot(h[0:N], wv, preferred_element_type=f32)
        s = jax.lax.dot_general(q, k, (((1,), (1,)), ((), ())),
                                preferred_element_type=f32) * INV_SQRT_D
        s = jnp.where(vm > 0.0, s, NEG)
        m = jnp.max(s, axis=1, keepdims=True)
        e = jnp.exp(s - m)
        p = e / jnp.sum(e, axis=1, keepdims=True)
        o = jnp.dot(p, v, preferred_element_type=f32)
        xq = x[qrows] + jnp.dot(o, wo, preferred_element_type=f32)
        h2 = ln(xq, g2, c2)
        a = jax.nn.gelu(jnp.dot(h2, w1, preferred_element_type=f32) + bb1)
        return xq + jnp.dot(a, w2, preferred_element_type=f32) + bb2

    wl0 = tuple(r[...] for r in wrefs[0:12])
    wl1 = tuple(r[...] for r in wrefs[12:24])
    x = attn_ffn(x, slice(0, NQ), wl0)
    xf = attn_ffn(x, slice(NQ - 16, NQ), wl1)
    rows_ref[0] = xf[7:9, :]


def _fused(ss, so, pos_tok, ws):
    full = lambda a: pl.BlockSpec(a.shape, lambda b: (0,) * a.ndim)
    return pl.pallas_call(
        _fused_body,
        grid=(B,),
        in_specs=[
            pl.BlockSpec((1, C, 1024), lambda b: (b, 0, 0)),
            pl.BlockSpec((1, C, 1024), lambda b: (b, 0, 0)),
            pl.BlockSpec((HW, D), lambda b: (0, 0)),
        ] + [full(w) for w in ws],
        out_specs=[
            pl.BlockSpec((1, 2, D), lambda b: (b, 0, 0)),
            pl.BlockSpec((1, 1, N), lambda b: (b, 0, 0)),
        ],
        out_shape=[
            jax.ShapeDtypeStruct((B, 2, D), jnp.float32),
            jax.ShapeDtypeStruct((B, 1, N), jnp.float32),
        ],
    )(ss, so, pos_tok, *ws)


def _heads_body(rows_ref, hv_ref, cs_ref, co_ref, ssc_ref, soc_ref,
                fw_ref, fb_ref, hw_ref, hb_ref, out_ref):
    f32 = jnp.float32
    hv = hv_ref[:, 0, :]
    all_valid = jnp.min(jnp.where(hv != 0.0, 1.0, 0.0), axis=1,
                        keepdims=True)
    hidden = (rows_ref[:, 0, :] * all_valid
              + rows_ref[:, 1, :] * (1.0 - all_valid))

    cs = cs_ref[...].reshape(B, 1)
    co = co_ref[...].reshape(B, 1)
    lane = jax.lax.broadcasted_iota(jnp.int32, (B, NC), 1)
    oh1 = (lane == cs).astype(f32)
    oh2 = (lane == co).astype(f32)
    hc = jnp.concatenate([hidden, oh1, oh2, ssc_ref[...], soc_ref[...]],
                         axis=1)

    pred = jnp.maximum(
        jnp.dot(hc, fw_ref[...], preferred_element_type=f32) + fb_ref[...],
        0.0)
    z = jnp.dot(pred, hw_ref[...], preferred_element_type=f32) + hb_ref[...]

    def lsm(seg):
        m = jnp.max(seg, axis=1, keepdims=True)
        lse = jnp.log(jnp.sum(jnp.exp(seg - m), axis=1, keepdims=True)) + m
        return seg - lse

    sup = lsm(z[:, 50:53])
    r1 = lsm(z[:, 0:15]) + sup[:, 0:1]
    r2 = lsm(z[:, 15:26]) + sup[:, 1:2]
    r3 = lsm(z[:, 26:50]) + sup[:, 2:3]
    conn = z[:, 53:54]
    out_ref[...] = jnp.concatenate(
        [r1, r2, r3, sup, conn, jnp.zeros((B, 128 - 54), f32)], axis=1)


def _heads(rows, hv, cs, co, ssc, soc, fw, fb, hwts, hb):
    args = (rows, hv, cs, co, ssc, soc, fw, fb, hwts, hb)
    full = lambda a: pl.BlockSpec(a.shape, lambda: (0,) * a.ndim)
    return pl.pallas_call(
        _heads_body,
        in_specs=[full(a) for a in args],
        out_specs=pl.BlockSpec((B, 128), lambda: (0, 0)),
        out_shape=jax.ShapeDtypeStruct((B, 128), jnp.float32),
    )(*args)


def kernel(src_sub, src_obj, cat_sub, cat_obj, scat_sub, scat_obj, params):
    f32 = jnp.float32
    pe = params['pe']
    pos_tok = jnp.transpose(pe[:16, :16], (1, 0, 2)).reshape(HW, 255)
    tail = (jnp.arange(D - 255)[None, :] == 1).astype(f32)
    pos_tok = jnp.concatenate(
        [pos_tok, jnp.broadcast_to(tail, (HW, D - 255))], axis=1)

    L = params['layers']
    keys = ('wq', 'wk', 'wv', 'wo', 'ln1_g', 'ln1_b', 'ln2_g', 'ln2_b',
            'w1', 'b1', 'w2', 'b2')
    ws = []
    for li in range(2):
        for key in keys:
            w = L[li][key]
            ws.append(w.reshape(1, -1) if w.ndim == 1 else w)

    rows, hv = _fused(src_sub.reshape(B, C, 1024),
                      src_obj.reshape(B, C, 1024), pos_tok, ws)

    hwts = jnp.concatenate([params['fc21_w'], params['fc22_w'],
                            params['fc23_w'], params['fc3_w'],
                            params['fc4_w']], axis=1)
    hb = jnp.concatenate([params['fc21_b'], params['fc22_b'],
                          params['fc23_b'], params['fc3_b'],
                          params['fc4_b']]).reshape(1, 54)

    out = _heads(rows, hv, cat_sub.reshape(1, B), cat_obj.reshape(1, B),
                 scat_sub, scat_obj, params['fc1_w'],
                 params['fc1_b'].reshape(1, D), hwts, hb)
    return (out[:, 0:15], out[:, 15:26], out[:, 26:50],
            out[:, 50:53], out[:, 53:54])
